# Initial kernel scaffold; baseline (speedup 1.0000x reference)
import jax
import jax.numpy as jnp
from jax.experimental import pallas as pl


def kernel(x, edge_index, W1, b1, W2, b2, Wm1, bm1, Wm2, bm2):
    raise NotImplementedError("write your pallas kernel here")


# speedup vs baseline: 21.9922x; 21.9922x over previous
import functools

import jax
import jax.numpy as jnp
from jax import lax
from jax.experimental import pallas as pl
from jax.experimental.pallas import tpu as pltpu
from jax.experimental.pallas import tpu_sc as plsc

N = 10000
E = 320000
C = 128

NC = 2
NS = 16
NW = NC * NS
B = 128
NCH = 80
EPW = NCH * B
EP = NW * EPW
SLAB = 16
NSLAB = NCH // SLAB
NPAD = 10240
RPT = NPAD // NS

_mesh = plsc.VectorSubcoreMesh(core_axis_name="c", subcore_axis_name="s")


@functools.partial(
    pl.kernel,
    out_type=jax.ShapeDtypeStruct((NC, NPAD, C), jnp.float32),
    mesh=_mesh,
    scratch_types=[
        pltpu.VMEM((NCH, B), jnp.int32),
        pltpu.VMEM((B, C), jnp.float32),
        pltpu.VMEM_SHARED((NPAD, C), jnp.float32),
    ],
)
def _sc_degree(dst_hbm, out_hbm, idx_v, ones_v, acc):
    c = lax.axis_index("c")
    s = lax.axis_index("s")
    wid = s * NC + c
    pltpu.sync_copy(dst_hbm.at[wid], idx_v)

    def zfill(i, _):
        ones_v[i // 8, pl.ds((i % 8) * 16, 16)] = jnp.zeros((16,), jnp.float32)
        return 0
    lax.fori_loop(0, B * 8, zfill, 0)
    for p in range(RPT // B):
        pltpu.sync_copy(ones_v, acc.at[pl.ds(s * RPT + p * B, B)])

    def fill(i, _):
        ones_v[i // 8, pl.ds((i % 8) * 16, 16)] = jnp.ones((16,), jnp.float32)
        return 0
    lax.fori_loop(0, B * 8, fill, 0)
    plsc.subcore_barrier()

    def chunk(j, _):
        pltpu.sync_copy(ones_v, acc.at[idx_v.at[j]], add=True)
        return 0
    lax.fori_loop(0, NCH, chunk, 0)

    plsc.subcore_barrier()
    pltpu.sync_copy(acc.at[pl.ds(s * RPT, RPT)],
                    out_hbm.at[c, pl.ds(s * RPT, RPT)])


def _agg_wait(y_hbm, idx_row, buf, sem):
    pltpu.make_async_copy(y_hbm.at[idx_row], buf, sem).wait()


@functools.partial(
    pl.kernel,
    out_type=jax.ShapeDtypeStruct((NC, NPAD, C), jnp.float32),
    mesh=_mesh,
    scratch_types=[
        pltpu.VMEM((SLAB, B), jnp.int32),
        pltpu.VMEM((SLAB, B), jnp.int32),
        pltpu.VMEM((B, C), jnp.float32),
        pltpu.VMEM((B, C), jnp.float32),
        pltpu.VMEM_SHARED((NPAD, C), jnp.float32),
        pltpu.SemaphoreType.DMA,
        pltpu.SemaphoreType.DMA,
    ],
)
def _sc_agg(y_hbm, src_hbm, dst_hbm, out_hbm,
            sidx, didx, rows0, rows1, acc, sem0, sem1):
    c = lax.axis_index("c")
    s = lax.axis_index("s")
    wid = s * NC + c

    def zfill(i, _):
        rows0[i // 8, pl.ds((i % 8) * 16, 16)] = jnp.zeros((16,), jnp.float32)
        return 0
    lax.fori_loop(0, B * 8, zfill, 0)
    for p in range(RPT // B):
        pltpu.sync_copy(rows0, acc.at[pl.ds(s * RPT + p * B, B)])
    plsc.subcore_barrier()

    for sl in range(NSLAB):
        pltpu.sync_copy(src_hbm.at[wid, pl.ds(sl * SLAB, SLAB)], sidx)
        pltpu.sync_copy(dst_hbm.at[wid, pl.ds(sl * SLAB, SLAB)], didx)

        pltpu.async_copy(y_hbm.at[sidx.at[0]], rows0, sem0)

        def body(i, _):
            j0 = 2 * i
            _agg_wait(y_hbm, sidx.at[j0], rows0, sem0)
            pltpu.async_copy(y_hbm.at[sidx.at[j0 + 1]], rows1, sem1)
            pltpu.sync_copy(rows0, acc.at[didx.at[j0]], add=True)
            _agg_wait(y_hbm, sidx.at[j0 + 1], rows1, sem1)
            pltpu.async_copy(y_hbm.at[sidx.at[j0 + 2]], rows0, sem0)
            pltpu.sync_copy(rows1, acc.at[didx.at[j0 + 1]], add=True)
            return 0
        lax.fori_loop(0, (SLAB - 2) // 2, body, 0)

        jt = SLAB - 2
        _agg_wait(y_hbm, sidx.at[jt], rows0, sem0)
        pltpu.async_copy(y_hbm.at[sidx.at[jt + 1]], rows1, sem1)
        pltpu.sync_copy(rows0, acc.at[didx.at[jt]], add=True)
        _agg_wait(y_hbm, sidx.at[jt + 1], rows1, sem1)
        pltpu.sync_copy(rows1, acc.at[didx.at[jt + 1]], add=True)

    plsc.subcore_barrier()
    pltpu.sync_copy(acc.at[pl.ds(s * RPT, RPT)],
                    out_hbm.at[c, pl.ds(s * RPT, RPT)])


RB = 400
GRID = (N // RB,)


def _rows(spec_cols):
    return pl.BlockSpec((RB, spec_cols), lambda i: (i, 0))


def _full(shape):
    return pl.BlockSpec(shape, lambda i: tuple(0 for _ in shape))


def _pair():
    return pl.BlockSpec((NC, RB, C), lambda i: (0, i, 0))


def _degspec():
    return pl.BlockSpec((NC, RB, C), lambda i: (0, i, 0))


def _ds_of(degp):
    deg = degp[0, :, 0:1] + degp[1, :, 0:1] + 1.0
    return lax.rsqrt(deg)


def _mm_body(x_ref, w_ref, o_ref):
    o_ref[...] = jnp.dot(x_ref[...], w_ref[...],
                         preferred_element_type=jnp.float32)


def _tc_mm(x, W):
    return pl.pallas_call(
        _mm_body,
        grid=GRID,
        in_specs=[_rows(C), _full((C, C))],
        out_specs=_rows(C),
        out_shape=jax.ShapeDtypeStruct((N, C), jnp.float32),
    )(x, W)


def _scale_body(xw_ref, degp_ref, y_ref):
    y_ref[...] = xw_ref[...] * _ds_of(degp_ref)


def _tc_scale(xw, degp):
    return pl.pallas_call(
        _scale_body,
        grid=GRID,
        in_specs=[_rows(C), _degspec()],
        out_specs=_rows(C),
        out_shape=jax.ShapeDtypeStruct((N, C), jnp.float32),
    )(xw, degp)


def _mid_body(aggp_ref, y1_ref, degp_ref, b1_ref, w2_ref, h_ref, y2_ref):
    ds = _ds_of(degp_ref)
    agg = aggp_ref[0] + aggp_ref[1]
    h = jnp.maximum(ds * (agg + y1_ref[...]) + b1_ref[...], 0.0)
    h_ref[...] = h
    y2_ref[...] = ds * jnp.dot(h, w2_ref[...],
                               preferred_element_type=jnp.float32)


def _tc_mid(aggp, y1, degp, b1, W2):
    return pl.pallas_call(
        _mid_body,
        grid=GRID,
        in_specs=[_pair(), _rows(C), _degspec(),
                  _full((1, C)), _full((C, C))],
        out_specs=[_rows(C), _rows(C)],
        out_shape=[jax.ShapeDtypeStruct((N, C), jnp.float32),
                   jax.ShapeDtypeStruct((N, C), jnp.float32)],
    )(aggp, y1, degp, b1, W2)


def _mask_body(h_ref, wm1_ref, bm1_ref, wm2_ref, bm2_ref, prob_ref, mask_ref):
    hidden = jnp.maximum(jnp.dot(h_ref[...], wm1_ref[...],
                                 preferred_element_type=jnp.float32)
                         + bm1_ref[...], 0.0)
    t = jnp.dot(hidden, wm2_ref[...],
                preferred_element_type=jnp.float32) + bm2_ref[...]
    p = jax.nn.sigmoid(t)
    prob_ref[...] = p
    mask_ref[...] = (p > 0.5).astype(jnp.float32)


def _tc_mask(h, Wm1, bm1, Wm2, bm2):
    return pl.pallas_call(
        _mask_body,
        grid=GRID,
        in_specs=[_rows(C), _full((C, 64)), _full((1, 64)),
                  _full((64, 1)), _full((1, 1))],
        out_specs=[_rows(1), _rows(1)],
        out_shape=[jax.ShapeDtypeStruct((N, 1), jnp.float32),
                   jax.ShapeDtypeStruct((N, 1), jnp.float32)],
    )(h, Wm1, bm1, Wm2, bm2)


def _final_body(aggp_ref, y2_ref, degp_ref, b2_ref, o_ref):
    ds = _ds_of(degp_ref)
    z = ds * (aggp_ref[0] + aggp_ref[1] + y2_ref[...]) + b2_ref[...]
    m = jnp.max(z, axis=1, keepdims=True)
    lse = jnp.log(jnp.sum(jnp.exp(z - m), axis=1, keepdims=True)) + m
    o_ref[...] = z - lse


def _tc_final(aggp, y2, degp, b2):
    return pl.pallas_call(
        _final_body,
        grid=GRID,
        in_specs=[_pair(), _rows(C), _degspec(), _full((1, C))],
        out_specs=_rows(C),
        out_shape=jax.ShapeDtypeStruct((N, C), jnp.float32),
    )(aggp, y2, degp, b2)


def kernel(x, edge_index, W1, b1, W2, b2, Wm1, bm1, Wm2, bm2):
    ei = edge_index.astype(jnp.int32)
    npd = EP - E
    pad_src = (jnp.arange(npd, dtype=jnp.int32) * 13) % N
    pad_dst = N + jnp.arange(npd, dtype=jnp.int32) % (NPAD - N)
    src = jnp.concatenate([ei[0], pad_src]).reshape(NW, NCH, B)
    dst = jnp.concatenate([ei[1], pad_dst]).reshape(NW, NCH, B)

    degp = _sc_degree(dst)
    xw1 = _tc_mm(x, W1)
    y1 = _tc_scale(xw1, degp)
    aggp1 = _sc_agg(y1, src, dst)
    h, y2 = _tc_mid(aggp1, y1, degp, b1.reshape(1, C), W2)
    aggp2 = _sc_agg(y2, src, dst)
    prob, mask = _tc_mask(h, Wm1, bm1.reshape(1, 64), Wm2, bm2.reshape(1, 1))
    logits = _tc_final(aggp2, y2, degp, b2.reshape(1, C))
    return logits, prob.reshape(N), mask.reshape(N)


# speedup vs baseline: 22.4511x; 1.0209x over previous
import functools

import jax
import jax.numpy as jnp
from jax import lax
from jax.experimental import pallas as pl
from jax.experimental.pallas import tpu as pltpu
from jax.experimental.pallas import tpu_sc as plsc

N = 10000
E = 320000
C = 128

NC = 2
NS = 16
NW = NC * NS
B = 128
NCH = 80
EPW = NCH * B
EP = NW * EPW
SLAB = 40
NSLAB = NCH // SLAB
NPAD = 10240
RPT = NPAD // NS

_mesh = plsc.VectorSubcoreMesh(core_axis_name="c", subcore_axis_name="s")


@functools.partial(
    pl.kernel,
    out_type=jax.ShapeDtypeStruct((NC, NPAD, C), jnp.float32),
    mesh=_mesh,
    scratch_types=[
        pltpu.VMEM((NCH, B), jnp.int32),
        pltpu.VMEM((B, C), jnp.float32),
        pltpu.VMEM_SHARED((NPAD, C), jnp.float32),
    ],
)
def _sc_degree(dst_hbm, out_hbm, idx_v, ones_v, acc):
    c = lax.axis_index("c")
    s = lax.axis_index("s")
    wid = s * NC + c
    pltpu.sync_copy(dst_hbm.at[wid], idx_v)

    def zfill(i, _):
        ones_v[i // 8, pl.ds((i % 8) * 16, 16)] = jnp.zeros((16,), jnp.float32)
        return 0
    lax.fori_loop(0, B * 8, zfill, 0)
    for p in range(RPT // B):
        pltpu.sync_copy(ones_v, acc.at[pl.ds(s * RPT + p * B, B)])

    def fill(i, _):
        ones_v[i // 8, pl.ds((i % 8) * 16, 16)] = jnp.ones((16,), jnp.float32)
        return 0
    lax.fori_loop(0, B * 8, fill, 0)
    plsc.subcore_barrier()

    def chunk(j, _):
        pltpu.sync_copy(ones_v, acc.at[idx_v.at[j]], add=True)
        return 0
    lax.fori_loop(0, NCH, chunk, 0)

    plsc.subcore_barrier()
    pltpu.sync_copy(acc.at[pl.ds(s * RPT, RPT)],
                    out_hbm.at[c, pl.ds(s * RPT, RPT)])


def _agg_wait(y_hbm, idx_row, buf, sem):
    pltpu.make_async_copy(y_hbm.at[idx_row], buf, sem).wait()


@functools.partial(
    pl.kernel,
    out_type=jax.ShapeDtypeStruct((NC, NPAD, C), jnp.float32),
    mesh=_mesh,
    scratch_types=[
        pltpu.VMEM((SLAB, B), jnp.int32),
        pltpu.VMEM((SLAB, B), jnp.int32),
        pltpu.VMEM((B, C), jnp.float32),
        pltpu.VMEM((B, C), jnp.float32),
        pltpu.VMEM_SHARED((NPAD, C), jnp.float32),
        pltpu.SemaphoreType.DMA,
        pltpu.SemaphoreType.DMA,
    ],
)
def _sc_agg(y_hbm, src_hbm, dst_hbm, out_hbm,
            sidx, didx, rows0, rows1, acc, sem0, sem1):
    c = lax.axis_index("c")
    s = lax.axis_index("s")
    wid = s * NC + c

    def zfill(i, _):
        rows0[i // 8, pl.ds((i % 8) * 16, 16)] = jnp.zeros((16,), jnp.float32)
        return 0
    lax.fori_loop(0, B * 8, zfill, 0)
    for p in range(RPT // B):
        pltpu.sync_copy(rows0, acc.at[pl.ds(s * RPT + p * B, B)])
    plsc.subcore_barrier()

    for sl in range(NSLAB):
        pltpu.sync_copy(src_hbm.at[wid, pl.ds(sl * SLAB, SLAB)], sidx)
        pltpu.sync_copy(dst_hbm.at[wid, pl.ds(sl * SLAB, SLAB)], didx)

        pltpu.async_copy(y_hbm.at[sidx.at[0]], rows0, sem0)

        def body(i, _):
            j0 = 2 * i
            _agg_wait(y_hbm, sidx.at[j0], rows0, sem0)
            pltpu.async_copy(y_hbm.at[sidx.at[j0 + 1]], rows1, sem1)
            pltpu.sync_copy(rows0, acc.at[didx.at[j0]], add=True)
            _agg_wait(y_hbm, sidx.at[j0 + 1], rows1, sem1)
            pltpu.async_copy(y_hbm.at[sidx.at[j0 + 2]], rows0, sem0)
            pltpu.sync_copy(rows1, acc.at[didx.at[j0 + 1]], add=True)
            return 0
        lax.fori_loop(0, (SLAB - 2) // 2, body, 0)

        jt = SLAB - 2
        _agg_wait(y_hbm, sidx.at[jt], rows0, sem0)
        pltpu.async_copy(y_hbm.at[sidx.at[jt + 1]], rows1, sem1)
        pltpu.sync_copy(rows0, acc.at[didx.at[jt]], add=True)
        _agg_wait(y_hbm, sidx.at[jt + 1], rows1, sem1)
        pltpu.sync_copy(rows1, acc.at[didx.at[jt + 1]], add=True)

    plsc.subcore_barrier()
    pltpu.sync_copy(acc.at[pl.ds(s * RPT, RPT)],
                    out_hbm.at[c, pl.ds(s * RPT, RPT)])


RB = 400
GRID = (N // RB,)


def _rows(spec_cols):
    return pl.BlockSpec((RB, spec_cols), lambda i: (i, 0))


def _full(shape):
    return pl.BlockSpec(shape, lambda i: tuple(0 for _ in shape))


def _pair():
    return pl.BlockSpec((NC, RB, C), lambda i: (0, i, 0))


def _degspec():
    return pl.BlockSpec((NC, RB, C), lambda i: (0, i, 0))


def _ds_of(degp):
    deg = degp[0, :, 0:1] + degp[1, :, 0:1] + 1.0
    return lax.rsqrt(deg)


def _scale_body(x_ref, w_ref, degp_ref, y_ref):
    xw = jnp.dot(x_ref[...], w_ref[...], preferred_element_type=jnp.float32)
    y_ref[...] = xw * _ds_of(degp_ref)


def _tc_scale(x, W, degp):
    return pl.pallas_call(
        _scale_body,
        grid=GRID,
        in_specs=[_rows(C), _full((C, C)), _degspec()],
        out_specs=_rows(C),
        out_shape=jax.ShapeDtypeStruct((N, C), jnp.float32),
    )(x, W, degp)


def _mid_body(aggp_ref, y1_ref, degp_ref, b1_ref, w2_ref,
              wm1_ref, bm1_ref, wm2_ref, bm2_ref,
              y2_ref, prob_ref, mask_ref):
    ds = _ds_of(degp_ref)
    agg = aggp_ref[0] + aggp_ref[1]
    h = jnp.maximum(ds * (agg + y1_ref[...]) + b1_ref[...], 0.0)
    y2_ref[...] = ds * jnp.dot(h, w2_ref[...],
                               preferred_element_type=jnp.float32)
    hidden = jnp.maximum(jnp.dot(h, wm1_ref[...],
                                 preferred_element_type=jnp.float32)
                         + bm1_ref[...], 0.0)
    t = jnp.dot(hidden, wm2_ref[...],
                preferred_element_type=jnp.float32) + bm2_ref[...]
    p = jax.nn.sigmoid(t)
    prob_ref[...] = p
    mask_ref[...] = (p > 0.5).astype(jnp.float32)


def _tc_mid(aggp, y1, degp, b1, W2, Wm1, bm1, Wm2, bm2):
    return pl.pallas_call(
        _mid_body,
        grid=GRID,
        in_specs=[_pair(), _rows(C), _degspec(),
                  _full((1, C)), _full((C, C)),
                  _full((C, 64)), _full((1, 64)),
                  _full((64, 1)), _full((1, 1))],
        out_specs=[_rows(C), _rows(1), _rows(1)],
        out_shape=[jax.ShapeDtypeStruct((N, C), jnp.float32),
                   jax.ShapeDtypeStruct((N, 1), jnp.float32),
                   jax.ShapeDtypeStruct((N, 1), jnp.float32)],
    )(aggp, y1, degp, b1, W2, Wm1, bm1, Wm2, bm2)


def _final_body(aggp_ref, y2_ref, degp_ref, b2_ref, o_ref):
    ds = _ds_of(degp_ref)
    z = ds * (aggp_ref[0] + aggp_ref[1] + y2_ref[...]) + b2_ref[...]
    m = jnp.max(z, axis=1, keepdims=True)
    lse = jnp.log(jnp.sum(jnp.exp(z - m), axis=1, keepdims=True)) + m
    o_ref[...] = z - lse


def _tc_final(aggp, y2, degp, b2):
    return pl.pallas_call(
        _final_body,
        grid=GRID,
        in_specs=[_pair(), _rows(C), _degspec(), _full((1, C))],
        out_specs=_rows(C),
        out_shape=jax.ShapeDtypeStruct((N, C), jnp.float32),
    )(aggp, y2, degp, b2)


def kernel(x, edge_index, W1, b1, W2, b2, Wm1, bm1, Wm2, bm2):
    ei = edge_index.astype(jnp.int32)
    npd = EP - E
    pad_src = (jnp.arange(npd, dtype=jnp.int32) * 13) % N
    pad_dst = N + jnp.arange(npd, dtype=jnp.int32) % (NPAD - N)
    src = jnp.concatenate([ei[0], pad_src]).reshape(NW, NCH, B)
    dst = jnp.concatenate([ei[1], pad_dst]).reshape(NW, NCH, B)

    degp = _sc_degree(dst)
    y1 = _tc_scale(x, W1, degp)
    aggp1 = _sc_agg(y1, src, dst)
    y2, prob, mask = _tc_mid(aggp1, y1, degp, b1.reshape(1, C), W2,
                             Wm1, bm1.reshape(1, 64), Wm2, bm2.reshape(1, 1))
    aggp2 = _sc_agg(y2, src, dst)
    logits = _tc_final(aggp2, y2, degp, b2.reshape(1, C))
    return logits, prob.reshape(N), mask.reshape(N)


# speedup vs baseline: 25.9129x; 1.1542x over previous
import functools

import jax
import jax.numpy as jnp
from jax import lax
from jax.experimental import pallas as pl
from jax.experimental.pallas import tpu as pltpu
from jax.experimental.pallas import tpu_sc as plsc

N = 10000
E = 320000
C = 128

NC = 2
NS = 16
NW = NC * NS
B = 64
NCH = 160
EPW = NCH * B
EP = NW * EPW
SLAB = 40
NSLAB = NCH // SLAB
NBUF = 4
BD = 128
NCHD = EPW // BD
NPAD = 10240
RPT = NPAD // NS

_mesh = plsc.VectorSubcoreMesh(core_axis_name="c", subcore_axis_name="s")


@functools.partial(
    pl.kernel,
    out_type=jax.ShapeDtypeStruct((NC, NPAD, C), jnp.float32),
    mesh=_mesh,
    scratch_types=[
        pltpu.VMEM((NCHD, BD), jnp.int32),
        pltpu.VMEM((BD, C), jnp.float32),
        pltpu.VMEM_SHARED((NPAD, C), jnp.float32),
        pltpu.SemaphoreType.DMA,
    ],
)
def _sc_degree(dst_hbm, out_hbm, idx_v, ones_v, acc, sem):
    c = lax.axis_index("c")
    s = lax.axis_index("s")
    wid = s * NC + c
    pltpu.sync_copy(dst_hbm.at[wid], idx_v)

    def zfill(i, _):
        ones_v[i // 8, pl.ds((i % 8) * 16, 16)] = jnp.zeros((16,), jnp.float32)
        return 0
    lax.fori_loop(0, BD * 8, zfill, 0)
    for p in range(RPT // BD):
        pltpu.sync_copy(ones_v, acc.at[pl.ds(s * RPT + p * BD, BD)])

    def fill(i, _):
        ones_v[i // 8, pl.ds((i % 8) * 16, 16)] = jnp.ones((16,), jnp.float32)
        return 0
    lax.fori_loop(0, BD * 8, fill, 0)
    plsc.subcore_barrier()

    RING = 8
    for j in range(RING):
        pltpu.async_copy(ones_v, acc.at[idx_v.at[j]], sem, add=True)

    def chunk(j, _):
        pltpu.make_async_copy(ones_v, acc.at[idx_v.at[j]], sem).wait()
        pltpu.async_copy(ones_v, acc.at[idx_v.at[j + RING]], sem, add=True)
        return 0
    lax.fori_loop(0, NCHD - RING, chunk, 0)

    def drain(j, _):
        pltpu.make_async_copy(ones_v, acc.at[idx_v.at[j]], sem).wait()
        return 0
    lax.fori_loop(NCHD - RING, NCHD, drain, 0)

    plsc.subcore_barrier()
    pltpu.sync_copy(acc.at[pl.ds(s * RPT, RPT)],
                    out_hbm.at[c, pl.ds(s * RPT, RPT)])


def _agg_wait(y_hbm, idx_row, buf, sem):
    pltpu.make_async_copy(y_hbm.at[idx_row], buf, sem).wait()


@functools.partial(
    pl.kernel,
    out_type=jax.ShapeDtypeStruct((NC, NPAD, C), jnp.float32),
    mesh=_mesh,
    scratch_types=[
        pltpu.VMEM((SLAB, B), jnp.int32),
        pltpu.VMEM((SLAB, B), jnp.int32),
        [pltpu.VMEM((B, C), jnp.float32)] * NBUF,
        pltpu.VMEM_SHARED((NPAD, C), jnp.float32),
        [pltpu.SemaphoreType.DMA] * NBUF,
    ],
)
def _sc_agg(y_hbm, src_hbm, dst_hbm, out_hbm,
            sidx, didx, rows, acc, sems):
    c = lax.axis_index("c")
    s = lax.axis_index("s")
    wid = s * NC + c

    def zfill(i, _):
        rows[0][i // 8, pl.ds((i % 8) * 16, 16)] = jnp.zeros((16,),
                                                             jnp.float32)
        return 0
    lax.fori_loop(0, B * 8, zfill, 0)
    for p in range(RPT // B):
        pltpu.sync_copy(rows[0], acc.at[pl.ds(s * RPT + p * B, B)])
    plsc.subcore_barrier()

    for sl in range(NSLAB):
        pltpu.sync_copy(src_hbm.at[wid, pl.ds(sl * SLAB, SLAB)], sidx)
        pltpu.sync_copy(dst_hbm.at[wid, pl.ds(sl * SLAB, SLAB)], didx)

        for b in range(NBUF):
            pltpu.async_copy(y_hbm.at[sidx.at[b]], rows[b], sems[b])

        def body(i, _):
            j0 = NBUF * i
            for b in range(NBUF):
                j = j0 + b
                _agg_wait(y_hbm, sidx.at[j], rows[b], sems[b])
                pltpu.sync_copy(rows[b], acc.at[didx.at[j]], add=True)
                pltpu.async_copy(y_hbm.at[sidx.at[j + NBUF]], rows[b],
                                 sems[b])
            return 0
        lax.fori_loop(0, SLAB // NBUF - 1, body, 0)

        for b in range(NBUF):
            j = SLAB - NBUF + b
            _agg_wait(y_hbm, sidx.at[j], rows[b], sems[b])
            pltpu.sync_copy(rows[b], acc.at[didx.at[j]], add=True)

    plsc.subcore_barrier()
    pltpu.sync_copy(acc.at[pl.ds(s * RPT, RPT)],
                    out_hbm.at[c, pl.ds(s * RPT, RPT)])


RB = 400
GRID = (N // RB,)


def _rows(spec_cols):
    return pl.BlockSpec((RB, spec_cols), lambda i: (i, 0))


def _full(shape):
    return pl.BlockSpec(shape, lambda i: tuple(0 for _ in shape))


def _pair():
    return pl.BlockSpec((NC, RB, C), lambda i: (0, i, 0))


def _degspec():
    return pl.BlockSpec((NC, RB, C), lambda i: (0, i, 0))


def _ds_of(degp):
    deg = degp[0, :, 0:1] + degp[1, :, 0:1] + 1.0
    return lax.rsqrt(deg)


def _scale_body(x_ref, w_ref, degp_ref, y_ref):
    xw = jnp.dot(x_ref[...], w_ref[...], preferred_element_type=jnp.float32)
    y_ref[...] = xw * _ds_of(degp_ref)


def _tc_scale(x, W, degp):
    return pl.pallas_call(
        _scale_body,
        grid=GRID,
        in_specs=[_rows(C), _full((C, C)), _degspec()],
        out_specs=_rows(C),
        out_shape=jax.ShapeDtypeStruct((N, C), jnp.float32),
    )(x, W, degp)


def _mid_body(aggp_ref, y1_ref, degp_ref, b1_ref, w2_ref,
              wm1_ref, bm1_ref, wm2_ref, bm2_ref,
              y2_ref, prob_ref, mask_ref):
    ds = _ds_of(degp_ref)
    agg = aggp_ref[0] + aggp_ref[1]
    h = jnp.maximum(ds * (agg + y1_ref[...]) + b1_ref[...], 0.0)
    y2_ref[...] = ds * jnp.dot(h, w2_ref[...],
                               preferred_element_type=jnp.float32)
    hidden = jnp.maximum(jnp.dot(h, wm1_ref[...],
                                 preferred_element_type=jnp.float32)
                         + bm1_ref[...], 0.0)
    t = jnp.dot(hidden, wm2_ref[...],
                preferred_element_type=jnp.float32) + bm2_ref[...]
    p = jax.nn.sigmoid(t)
    prob_ref[...] = p
    mask_ref[...] = (p > 0.5).astype(jnp.float32)


def _tc_mid(aggp, y1, degp, b1, W2, Wm1, bm1, Wm2, bm2):
    return pl.pallas_call(
        _mid_body,
        grid=GRID,
        in_specs=[_pair(), _rows(C), _degspec(),
                  _full((1, C)), _full((C, C)),
                  _full((C, 64)), _full((1, 64)),
                  _full((64, 1)), _full((1, 1))],
        out_specs=[_rows(C), _rows(1), _rows(1)],
        out_shape=[jax.ShapeDtypeStruct((N, C), jnp.float32),
                   jax.ShapeDtypeStruct((N, 1), jnp.float32),
                   jax.ShapeDtypeStruct((N, 1), jnp.float32)],
    )(aggp, y1, degp, b1, W2, Wm1, bm1, Wm2, bm2)


def _final_body(aggp_ref, y2_ref, degp_ref, b2_ref, o_ref):
    ds = _ds_of(degp_ref)
    z = ds * (aggp_ref[0] + aggp_ref[1] + y2_ref[...]) + b2_ref[...]
    m = jnp.max(z, axis=1, keepdims=True)
    lse = jnp.log(jnp.sum(jnp.exp(z - m), axis=1, keepdims=True)) + m
    o_ref[...] = z - lse


def _tc_final(aggp, y2, degp, b2):
    return pl.pallas_call(
        _final_body,
        grid=GRID,
        in_specs=[_pair(), _rows(C), _degspec(), _full((1, C))],
        out_specs=_rows(C),
        out_shape=jax.ShapeDtypeStruct((N, C), jnp.float32),
    )(aggp, y2, degp, b2)


def kernel(x, edge_index, W1, b1, W2, b2, Wm1, bm1, Wm2, bm2):
    ei = edge_index.astype(jnp.int32)
    npd = EP - E
    pad_src = (jnp.arange(npd, dtype=jnp.int32) * 13) % N
    pad_dst = N + jnp.arange(npd, dtype=jnp.int32) % (NPAD - N)
    src = jnp.concatenate([ei[0], pad_src]).reshape(NW, NCH, B)
    dstflat = jnp.concatenate([ei[1], pad_dst])
    dst = dstflat.reshape(NW, NCH, B)
    dstD = dstflat.reshape(NW, NCHD, BD)

    degp = _sc_degree(dstD)
    y1 = _tc_scale(x, W1, degp)
    aggp1 = _sc_agg(y1, src, dst)
    y2, prob, mask = _tc_mid(aggp1, y1, degp, b1.reshape(1, C), W2,
                             Wm1, bm1.reshape(1, 64), Wm2, bm2.reshape(1, 1))
    aggp2 = _sc_agg(y2, src, dst)
    logits = _tc_final(aggp2, y2, degp, b2.reshape(1, C))
    return logits, prob.reshape(N), mask.reshape(N)


# speedup vs baseline: 27.0348x; 1.0433x over previous
import functools

import jax
import jax.numpy as jnp
from jax import lax
from jax.experimental import pallas as pl
from jax.experimental.pallas import tpu as pltpu
from jax.experimental.pallas import tpu_sc as plsc

N = 10000
E = 320000
C = 128

NC = 2
NS = 16
NW = NC * NS
B = 64
NCH = 160
EPW = NCH * B
EP = NW * EPW
SLAB = 16
NSLAB = NCH // SLAB
NBUF = 4
BD = 128
NCHD = EPW // BD
NPAD = 10240
RPT = NPAD // NS

_mesh = plsc.VectorSubcoreMesh(core_axis_name="c", subcore_axis_name="s")


@functools.partial(
    pl.kernel,
    out_type=jax.ShapeDtypeStruct((NC, NPAD, C), jnp.float32),
    mesh=_mesh,
    scratch_types=[
        pltpu.VMEM((NCHD, BD), jnp.int32),
        pltpu.VMEM((BD, C), jnp.float32),
        pltpu.VMEM_SHARED((NPAD, C), jnp.float32),
        pltpu.SemaphoreType.DMA,
    ],
)
def _sc_degree(dst_hbm, out_hbm, idx_v, ones_v, acc, sem):
    c = lax.axis_index("c")
    s = lax.axis_index("s")
    wid = s * NC + c
    pltpu.sync_copy(dst_hbm.at[wid], idx_v)

    def zfill(i, _):
        ones_v[i // 8, pl.ds((i % 8) * 16, 16)] = jnp.zeros((16,), jnp.float32)
        return 0
    lax.fori_loop(0, BD * 8, zfill, 0)
    for p in range(RPT // BD):
        pltpu.sync_copy(ones_v, acc.at[pl.ds(s * RPT + p * BD, BD)])

    def fill(i, _):
        ones_v[i // 8, pl.ds((i % 8) * 16, 16)] = jnp.ones((16,), jnp.float32)
        return 0
    lax.fori_loop(0, BD * 8, fill, 0)
    plsc.subcore_barrier()

    RING = 8
    for j in range(RING):
        pltpu.async_copy(ones_v, acc.at[idx_v.at[j]], sem, add=True)

    def chunk(j, _):
        pltpu.make_async_copy(ones_v, acc.at[idx_v.at[j]], sem).wait()
        pltpu.async_copy(ones_v, acc.at[idx_v.at[j + RING]], sem, add=True)
        return 0
    lax.fori_loop(0, NCHD - RING, chunk, 0)

    def drain(j, _):
        pltpu.make_async_copy(ones_v, acc.at[idx_v.at[j]], sem).wait()
        return 0
    lax.fori_loop(NCHD - RING, NCHD, drain, 0)

    plsc.subcore_barrier()
    pltpu.sync_copy(acc.at[pl.ds(s * RPT, RPT)],
                    out_hbm.at[c, pl.ds(s * RPT, RPT)])


def _agg_wait(y_hbm, idx_row, buf, sem):
    pltpu.make_async_copy(y_hbm.at[idx_row], buf, sem).wait()


@functools.partial(
    pl.kernel,
    out_type=jax.ShapeDtypeStruct((NC, NPAD, C), jnp.float32),
    mesh=_mesh,
    scratch_types=[
        [pltpu.VMEM((SLAB, B), jnp.int32)] * 2,
        [pltpu.VMEM((SLAB, B), jnp.int32)] * 2,
        [pltpu.VMEM((B, C), jnp.float32)] * NBUF,
        pltpu.VMEM_SHARED((NPAD, C), jnp.float32),
        [pltpu.SemaphoreType.DMA] * NBUF,
        pltpu.SemaphoreType.DMA,
    ],
)
def _sc_agg(y_hbm, src_hbm, dst_hbm, out_hbm,
            sidx, didx, rows, acc, sems, semi):
    c = lax.axis_index("c")
    s = lax.axis_index("s")
    wid = s * NC + c

    def zfill(i, _):
        rows[0][i // 8, pl.ds((i % 8) * 16, 16)] = jnp.zeros((16,),
                                                             jnp.float32)
        return 0
    lax.fori_loop(0, B * 8, zfill, 0)
    for p in range(RPT // B):
        pltpu.sync_copy(rows[0], acc.at[pl.ds(s * RPT + p * B, B)])
    plsc.subcore_barrier()

    def _stage(sl, asyncly):
        p = sl % 2
        src_sl = src_hbm.at[wid, pl.ds(sl * SLAB, SLAB)]
        dst_sl = dst_hbm.at[wid, pl.ds(sl * SLAB, SLAB)]
        if asyncly:
            pltpu.async_copy(src_sl, sidx[p], semi)
            pltpu.async_copy(dst_sl, didx[p], semi)
        else:
            pltpu.sync_copy(src_sl, sidx[p])
            pltpu.sync_copy(dst_sl, didx[p])

    def _stage_wait(sl):
        p = sl % 2
        pltpu.make_async_copy(src_hbm.at[wid, pl.ds(sl * SLAB, SLAB)],
                              sidx[p], semi).wait()
        pltpu.make_async_copy(dst_hbm.at[wid, pl.ds(sl * SLAB, SLAB)],
                              didx[p], semi).wait()

    def _srow(g):
        return sidx[(g // SLAB) % 2].at[g % SLAB]

    _stage(0, False)
    for g in range(NBUF):
        pltpu.async_copy(y_hbm.at[_srow(g)], rows[g % NBUF], sems[g % NBUF])

    for sl in range(NSLAB):
        if sl + 1 < NSLAB:
            _stage(sl + 1, True)
        for j in range(SLAB):
            g = sl * SLAB + j
            b = g % NBUF
            if j == SLAB - NBUF and sl + 1 < NSLAB:
                _stage_wait(sl + 1)
            _agg_wait(y_hbm, _srow(g), rows[b], sems[b])
            pltpu.sync_copy(rows[b], acc.at[didx[sl % 2].at[j]], add=True)
            if g + NBUF < NCH:
                pltpu.async_copy(y_hbm.at[_srow(g + NBUF)], rows[b], sems[b])

    plsc.subcore_barrier()
    pltpu.sync_copy(acc.at[pl.ds(s * RPT, RPT)],
                    out_hbm.at[c, pl.ds(s * RPT, RPT)])


RB = 400
GRID = (N // RB,)


def _rows(spec_cols):
    return pl.BlockSpec((RB, spec_cols), lambda i: (i, 0))


def _full(shape):
    return pl.BlockSpec(shape, lambda i: tuple(0 for _ in shape))


def _pair():
    return pl.BlockSpec((NC, RB, C), lambda i: (0, i, 0))


def _degspec():
    return pl.BlockSpec((NC, RB, C), lambda i: (0, i, 0))


def _ds_of(degp):
    deg = degp[0, :, 0:1] + degp[1, :, 0:1] + 1.0
    return lax.rsqrt(deg)


def _scale_body(x_ref, w_ref, degp_ref, y_ref):
    xw = jnp.dot(x_ref[...], w_ref[...], preferred_element_type=jnp.float32)
    y_ref[...] = xw * _ds_of(degp_ref)


def _tc_scale(x, W, degp):
    return pl.pallas_call(
        _scale_body,
        grid=GRID,
        in_specs=[_rows(C), _full((C, C)), _degspec()],
        out_specs=_rows(C),
        out_shape=jax.ShapeDtypeStruct((N, C), jnp.float32),
    )(x, W, degp)


def _mid_body(aggp_ref, y1_ref, degp_ref, b1_ref, w2_ref,
              wm1_ref, bm1_ref, wm2_ref, bm2_ref,
              y2_ref, prob_ref, mask_ref):
    ds = _ds_of(degp_ref)
    agg = aggp_ref[0] + aggp_ref[1]
    h = jnp.maximum(ds * (agg + y1_ref[...]) + b1_ref[...], 0.0)
    y2_ref[...] = ds * jnp.dot(h, w2_ref[...],
                               preferred_element_type=jnp.float32)
    hidden = jnp.maximum(jnp.dot(h, wm1_ref[...],
                                 preferred_element_type=jnp.float32)
                         + bm1_ref[...], 0.0)
    t = jnp.dot(hidden, wm2_ref[...],
                preferred_element_type=jnp.float32) + bm2_ref[...]
    p = jax.nn.sigmoid(t)
    prob_ref[...] = p
    mask_ref[...] = (p > 0.5).astype(jnp.float32)


def _tc_mid(aggp, y1, degp, b1, W2, Wm1, bm1, Wm2, bm2):
    return pl.pallas_call(
        _mid_body,
        grid=GRID,
        in_specs=[_pair(), _rows(C), _degspec(),
                  _full((1, C)), _full((C, C)),
                  _full((C, 64)), _full((1, 64)),
                  _full((64, 1)), _full((1, 1))],
        out_specs=[_rows(C), _rows(1), _rows(1)],
        out_shape=[jax.ShapeDtypeStruct((N, C), jnp.float32),
                   jax.ShapeDtypeStruct((N, 1), jnp.float32),
                   jax.ShapeDtypeStruct((N, 1), jnp.float32)],
    )(aggp, y1, degp, b1, W2, Wm1, bm1, Wm2, bm2)


def _final_body(aggp_ref, y2_ref, degp_ref, b2_ref, o_ref):
    ds = _ds_of(degp_ref)
    z = ds * (aggp_ref[0] + aggp_ref[1] + y2_ref[...]) + b2_ref[...]
    m = jnp.max(z, axis=1, keepdims=True)
    lse = jnp.log(jnp.sum(jnp.exp(z - m), axis=1, keepdims=True)) + m
    o_ref[...] = z - lse


def _tc_final(aggp, y2, degp, b2):
    return pl.pallas_call(
        _final_body,
        grid=GRID,
        in_specs=[_pair(), _rows(C), _degspec(), _full((1, C))],
        out_specs=_rows(C),
        out_shape=jax.ShapeDtypeStruct((N, C), jnp.float32),
    )(aggp, y2, degp, b2)


def kernel(x, edge_index, W1, b1, W2, b2, Wm1, bm1, Wm2, bm2):
    ei = edge_index.astype(jnp.int32)
    npd = EP - E
    pad_src = (jnp.arange(npd, dtype=jnp.int32) * 13) % N
    pad_dst = N + jnp.arange(npd, dtype=jnp.int32) % (NPAD - N)
    src = jnp.concatenate([ei[0], pad_src]).reshape(NW, NCH, B)
    dstflat = jnp.concatenate([ei[1], pad_dst])
    dst = dstflat.reshape(NW, NCH, B)
    dstD = dstflat.reshape(NW, NCHD, BD)

    degp = _sc_degree(dstD)
    y1 = _tc_scale(x, W1, degp)
    aggp1 = _sc_agg(y1, src, dst)
    y2, prob, mask = _tc_mid(aggp1, y1, degp, b1.reshape(1, C), W2,
                             Wm1, bm1.reshape(1, 64), Wm2, bm2.reshape(1, 1))
    aggp2 = _sc_agg(y2, src, dst)
    logits = _tc_final(aggp2, y2, degp, b2.reshape(1, C))
    return logits, prob.reshape(N), mask.reshape(N)


# speedup vs baseline: 27.1850x; 1.0056x over previous
import functools

import jax
import jax.numpy as jnp
from jax import lax
from jax.experimental import pallas as pl
from jax.experimental.pallas import tpu as pltpu
from jax.experimental.pallas import tpu_sc as plsc

N = 10000
E = 320000
C = 128

NC = 2
NS = 16
NW = NC * NS
B = 80
NCH = 128
EPW = NCH * B
EP = NW * EPW
SLAB = 8
NSLAB = NCH // SLAB
NBUF = 4
BD = 128
NCHD = EPW // BD
NPAD = 10240
RPT = NPAD // NS

_mesh = plsc.VectorSubcoreMesh(core_axis_name="c", subcore_axis_name="s")


@functools.partial(
    pl.kernel,
    out_type=jax.ShapeDtypeStruct((NC, NPAD, C), jnp.float32),
    mesh=_mesh,
    scratch_types=[
        pltpu.VMEM((NCHD, BD), jnp.int32),
        pltpu.VMEM((BD, C), jnp.float32),
        pltpu.VMEM_SHARED((NPAD, C), jnp.float32),
        pltpu.SemaphoreType.DMA,
    ],
)
def _sc_degree(dst_hbm, out_hbm, idx_v, ones_v, acc, sem):
    c = lax.axis_index("c")
    s = lax.axis_index("s")
    wid = s * NC + c
    pltpu.sync_copy(dst_hbm.at[wid], idx_v)

    def zfill(i, _):
        ones_v[i // 8, pl.ds((i % 8) * 16, 16)] = jnp.zeros((16,), jnp.float32)
        return 0
    lax.fori_loop(0, BD * 8, zfill, 0)
    for p in range(RPT // BD):
        pltpu.sync_copy(ones_v, acc.at[pl.ds(s * RPT + p * BD, BD)])

    def fill(i, _):
        ones_v[i // 8, pl.ds((i % 8) * 16, 16)] = jnp.ones((16,), jnp.float32)
        return 0
    lax.fori_loop(0, BD * 8, fill, 0)
    plsc.subcore_barrier()

    RING = 8
    for j in range(RING):
        pltpu.async_copy(ones_v, acc.at[idx_v.at[j]], sem, add=True)

    def chunk(j, _):
        pltpu.make_async_copy(ones_v, acc.at[idx_v.at[j]], sem).wait()
        pltpu.async_copy(ones_v, acc.at[idx_v.at[j + RING]], sem, add=True)
        return 0
    lax.fori_loop(0, NCHD - RING, chunk, 0)

    def drain(j, _):
        pltpu.make_async_copy(ones_v, acc.at[idx_v.at[j]], sem).wait()
        return 0
    lax.fori_loop(NCHD - RING, NCHD, drain, 0)

    plsc.subcore_barrier()
    pltpu.sync_copy(acc.at[pl.ds(s * RPT, RPT)],
                    out_hbm.at[c, pl.ds(s * RPT, RPT)])


def _agg_wait(y_hbm, idx_row, buf, sem):
    pltpu.make_async_copy(y_hbm.at[idx_row], buf, sem).wait()


@functools.partial(
    pl.kernel,
    out_type=jax.ShapeDtypeStruct((NC, NPAD, C), jnp.float32),
    mesh=_mesh,
    scratch_types=[
        [pltpu.VMEM((SLAB, B), jnp.int32)] * 2,
        [pltpu.VMEM((SLAB, B), jnp.int32)] * 2,
        [pltpu.VMEM((B, C), jnp.float32)] * NBUF,
        pltpu.VMEM_SHARED((NPAD, C), jnp.float32),
        [pltpu.SemaphoreType.DMA] * NBUF,
        pltpu.SemaphoreType.DMA,
    ],
)
def _sc_agg(y_hbm, src_hbm, dst_hbm, out_hbm,
            sidx, didx, rows, acc, sems, semi):
    c = lax.axis_index("c")
    s = lax.axis_index("s")
    wid = s * NC + c

    def zfill(i, _):
        rows[0][i // 8, pl.ds((i % 8) * 16, 16)] = jnp.zeros((16,),
                                                             jnp.float32)
        return 0
    lax.fori_loop(0, B * 8, zfill, 0)
    for p in range(RPT // B):
        pltpu.sync_copy(rows[0], acc.at[pl.ds(s * RPT + p * B, B)])
    plsc.subcore_barrier()

    def _stage(sl, asyncly):
        p = sl % 2
        src_sl = src_hbm.at[wid, pl.ds(sl * SLAB, SLAB)]
        dst_sl = dst_hbm.at[wid, pl.ds(sl * SLAB, SLAB)]
        if asyncly:
            pltpu.async_copy(src_sl, sidx[p], semi)
            pltpu.async_copy(dst_sl, didx[p], semi)
        else:
            pltpu.sync_copy(src_sl, sidx[p])
            pltpu.sync_copy(dst_sl, didx[p])

    def _stage_wait(sl):
        p = sl % 2
        pltpu.make_async_copy(src_hbm.at[wid, pl.ds(sl * SLAB, SLAB)],
                              sidx[p], semi).wait()
        pltpu.make_async_copy(dst_hbm.at[wid, pl.ds(sl * SLAB, SLAB)],
                              didx[p], semi).wait()

    def _srow(g):
        return sidx[(g // SLAB) % 2].at[g % SLAB]

    _stage(0, False)
    for g in range(NBUF):
        pltpu.async_copy(y_hbm.at[_srow(g)], rows[g % NBUF], sems[g % NBUF])

    for sl in range(NSLAB):
        if sl + 1 < NSLAB:
            _stage(sl + 1, True)
        for j in range(SLAB):
            g = sl * SLAB + j
            b = g % NBUF
            if j == SLAB - NBUF and sl + 1 < NSLAB:
                _stage_wait(sl + 1)
            _agg_wait(y_hbm, _srow(g), rows[b], sems[b])
            pltpu.sync_copy(rows[b], acc.at[didx[sl % 2].at[j]], add=True)
            if g + NBUF < NCH:
                pltpu.async_copy(y_hbm.at[_srow(g + NBUF)], rows[b], sems[b])

    plsc.subcore_barrier()
    pltpu.sync_copy(acc.at[pl.ds(s * RPT, RPT)],
                    out_hbm.at[c, pl.ds(s * RPT, RPT)])


RB = 400
GRID = (N // RB,)


def _rows(spec_cols):
    return pl.BlockSpec((RB, spec_cols), lambda i: (i, 0))


def _full(shape):
    return pl.BlockSpec(shape, lambda i: tuple(0 for _ in shape))


def _pair():
    return pl.BlockSpec((NC, RB, C), lambda i: (0, i, 0))


def _degspec():
    return pl.BlockSpec((NC, RB, C), lambda i: (0, i, 0))


def _ds_of(degp):
    deg = degp[0, :, 0:1] + degp[1, :, 0:1] + 1.0
    return lax.rsqrt(deg)


def _scale_body(x_ref, w_ref, degp_ref, y_ref):
    xw = jnp.dot(x_ref[...], w_ref[...], preferred_element_type=jnp.float32)
    y_ref[...] = xw * _ds_of(degp_ref)


def _tc_scale(x, W, degp):
    return pl.pallas_call(
        _scale_body,
        grid=GRID,
        in_specs=[_rows(C), _full((C, C)), _degspec()],
        out_specs=_rows(C),
        out_shape=jax.ShapeDtypeStruct((N, C), jnp.float32),
    )(x, W, degp)


def _mid_body(aggp_ref, y1_ref, degp_ref, b1_ref, w2_ref,
              wm1_ref, bm1_ref, wm2_ref, bm2_ref,
              y2_ref, prob_ref, mask_ref):
    ds = _ds_of(degp_ref)
    agg = aggp_ref[0] + aggp_ref[1]
    h = jnp.maximum(ds * (agg + y1_ref[...]) + b1_ref[...], 0.0)
    y2_ref[...] = ds * jnp.dot(h, w2_ref[...],
                               preferred_element_type=jnp.float32)
    hidden = jnp.maximum(jnp.dot(h, wm1_ref[...],
                                 preferred_element_type=jnp.float32)
                         + bm1_ref[...], 0.0)
    t = jnp.dot(hidden, wm2_ref[...],
                preferred_element_type=jnp.float32) + bm2_ref[...]
    p = jax.nn.sigmoid(t)
    prob_ref[...] = p
    mask_ref[...] = (p > 0.5).astype(jnp.float32)


def _tc_mid(aggp, y1, degp, b1, W2, Wm1, bm1, Wm2, bm2):
    return pl.pallas_call(
        _mid_body,
        grid=GRID,
        in_specs=[_pair(), _rows(C), _degspec(),
                  _full((1, C)), _full((C, C)),
                  _full((C, 64)), _full((1, 64)),
                  _full((64, 1)), _full((1, 1))],
        out_specs=[_rows(C), _rows(1), _rows(1)],
        out_shape=[jax.ShapeDtypeStruct((N, C), jnp.float32),
                   jax.ShapeDtypeStruct((N, 1), jnp.float32),
                   jax.ShapeDtypeStruct((N, 1), jnp.float32)],
    )(aggp, y1, degp, b1, W2, Wm1, bm1, Wm2, bm2)


def _final_body(aggp_ref, y2_ref, degp_ref, b2_ref, o_ref):
    ds = _ds_of(degp_ref)
    z = ds * (aggp_ref[0] + aggp_ref[1] + y2_ref[...]) + b2_ref[...]
    m = jnp.max(z, axis=1, keepdims=True)
    lse = jnp.log(jnp.sum(jnp.exp(z - m), axis=1, keepdims=True)) + m
    o_ref[...] = z - lse


def _tc_final(aggp, y2, degp, b2):
    return pl.pallas_call(
        _final_body,
        grid=GRID,
        in_specs=[_pair(), _rows(C), _degspec(), _full((1, C))],
        out_specs=_rows(C),
        out_shape=jax.ShapeDtypeStruct((N, C), jnp.float32),
    )(aggp, y2, degp, b2)


def kernel(x, edge_index, W1, b1, W2, b2, Wm1, bm1, Wm2, bm2):
    ei = edge_index.astype(jnp.int32)
    npd = EP - E
    pad_src = (jnp.arange(npd, dtype=jnp.int32) * 13) % N
    pad_dst = N + jnp.arange(npd, dtype=jnp.int32) % (NPAD - N)
    src = jnp.concatenate([ei[0], pad_src]).reshape(NW, NCH, B)
    dstflat = jnp.concatenate([ei[1], pad_dst])
    dst = dstflat.reshape(NW, NCH, B)
    dstD = dstflat.reshape(NW, NCHD, BD)

    degp = _sc_degree(dstD)
    y1 = _tc_scale(x, W1, degp)
    aggp1 = _sc_agg(y1, src, dst)
    y2, prob, mask = _tc_mid(aggp1, y1, degp, b1.reshape(1, C), W2,
                             Wm1, bm1.reshape(1, 64), Wm2, bm2.reshape(1, 1))
    aggp2 = _sc_agg(y2, src, dst)
    logits = _tc_final(aggp2, y2, degp, b2.reshape(1, C))
    return logits, prob.reshape(N), mask.reshape(N)
